# baseline (speedup 1.0000x reference)
import functools

import jax
import jax.numpy as jnp
from jax import lax
from jax.experimental import pallas as pl
from jax.experimental.pallas import tpu as pltpu
from jax.experimental.pallas import tpu_sc as plsc

_NUM_BUCKETS = 100000
_DIM = 100
_K = 2
_PRIME0, _PRIME1 = 31, 1009
_OFF0, _OFF1 = 7, 433
_BATCH = 16384
_OUT = 300

_NC = 2
_NS = 16
_NW = _NC * _NS
_BPW = _BATCH // _NW
_LANES = 16


def _sc_gather(x, table, importance):
    mesh = plsc.VectorSubcoreMesh(core_axis_name="c", subcore_axis_name="s")

    @functools.partial(
        pl.kernel,
        mesh=mesh,
        compiler_params=pltpu.CompilerParams(use_tc_tiling_on_sc=False),
        out_type=(
            jax.ShapeDtypeStruct((_BATCH, _DIM), jnp.float32),
            jax.ShapeDtypeStruct((_BATCH, _DIM), jnp.float32),
            jax.ShapeDtypeStruct((_BATCH, _K), jnp.float32),
        ),
        scratch_types=[
            pltpu.VMEM((_BPW,), jnp.int32),
            pltpu.VMEM((_BPW,), jnp.int32),
            pltpu.VMEM((_BPW,), jnp.int32),
            pltpu.VMEM((_BPW, _DIM), jnp.float32),
            pltpu.VMEM((_BPW, _DIM), jnp.float32),
            pltpu.VMEM((_BPW, _K), jnp.float32),
            pltpu.SemaphoreType.DMA,
        ],
    )
    def gather_kernel(x_hbm, table_hbm, imp_hbm, t0_hbm, t1_hbm, impo_hbm,
                      x_v, h0_v, h1_v, t0_v, t1_v, imp_v, sem):
        wid = lax.axis_index("s") * _NC + lax.axis_index("c")
        base = wid * _BPW
        pltpu.sync_copy(x_hbm.at[pl.ds(base, _BPW)], x_v)

        def hash_body(i, carry):
            xx = x_v[pl.ds(i * _LANES, _LANES)]
            h0_v[pl.ds(i * _LANES, _LANES)] = (xx * _PRIME0 + _OFF0) % _NUM_BUCKETS
            h1_v[pl.ds(i * _LANES, _LANES)] = (xx * _PRIME1 + _OFF1) % _NUM_BUCKETS
            return carry

        lax.fori_loop(0, _BPW // _LANES, hash_body, 0)

        cp0 = pltpu.async_copy(table_hbm.at[h0_v], t0_v, sem)
        cp1 = pltpu.async_copy(table_hbm.at[h1_v], t1_v, sem)
        cp2 = pltpu.async_copy(imp_hbm.at[x_v], imp_v, sem)
        cp0.wait()
        cp1.wait()
        cp2.wait()

        pltpu.sync_copy(t0_v, t0_hbm.at[pl.ds(base, _BPW)])
        pltpu.sync_copy(t1_v, t1_hbm.at[pl.ds(base, _BPW)])
        pltpu.sync_copy(imp_v, impo_hbm.at[pl.ds(base, _BPW)])

    return gather_kernel(x, table, importance)


_TC_BLK = 1024


def _tc_head_body(t0_ref, t1_ref, imp_ref, w_ref, b_ref, o_ref):
    imp = imp_ref[...]
    emb = imp[:, 0:1] * t0_ref[...] + imp[:, 1:2] * t1_ref[...]
    out = jnp.dot(emb, w_ref[...], preferred_element_type=jnp.float32)
    out = out + b_ref[...]
    m = jnp.max(out, axis=1, keepdims=True)
    e = jnp.exp(out - m)
    s = jnp.sum(e, axis=1, keepdims=True)
    o_ref[...] = out - m - jnp.log(s)


def _tc_head(t0, t1, imp, fc_w, fc_b):
    nblk = _BATCH // _TC_BLK
    return pl.pallas_call(
        _tc_head_body,
        grid=(nblk,),
        in_specs=[
            pl.BlockSpec((_TC_BLK, _DIM), lambda i: (i, 0)),
            pl.BlockSpec((_TC_BLK, _DIM), lambda i: (i, 0)),
            pl.BlockSpec((_TC_BLK, _K), lambda i: (i, 0)),
            pl.BlockSpec((_DIM, _OUT), lambda i: (0, 0)),
            pl.BlockSpec((1, _OUT), lambda i: (0, 0)),
        ],
        out_specs=pl.BlockSpec((_TC_BLK, _OUT), lambda i: (i, 0)),
        out_shape=jax.ShapeDtypeStruct((_BATCH, _OUT), jnp.float32),
    )(t0, t1, imp, fc_w, fc_b)


def kernel(x, table, importance, fc_w, fc_b):
    t0, t1, imp = _sc_gather(x, table, importance)
    return _tc_head(t0, t1, imp, fc_w.astype(jnp.float32),
                    fc_b.reshape(1, _OUT).astype(jnp.float32))


# speedup vs baseline: 8.0748x; 8.0748x over previous
import functools

import jax
import jax.numpy as jnp
from jax import lax
from jax.experimental import pallas as pl
from jax.experimental.pallas import tpu as pltpu
from jax.experimental.pallas import tpu_sc as plsc

_NUM_BUCKETS = 100000
_DIM = 100
_DIMP = 128
_PRIME0, _PRIME1 = 31, 1009
_OFF0, _OFF1 = 7, 433
_BATCH = 16384
_OUT = 300

_NC = 2
_NS = 16
_NW = _NC * _NS
_BPW = _BATCH // _NW
_SUB = 256
_LANES = 16


def _sc_gather(x, table_p, imp0, imp1):
    mesh = plsc.VectorSubcoreMesh(core_axis_name="c", subcore_axis_name="s")

    @functools.partial(
        pl.kernel,
        mesh=mesh,
        out_type=(
            jax.ShapeDtypeStruct((_BATCH, _DIMP), jnp.float32),
            jax.ShapeDtypeStruct((_BATCH, _DIMP), jnp.float32),
            jax.ShapeDtypeStruct((_BATCH,), jnp.float32),
            jax.ShapeDtypeStruct((_BATCH,), jnp.float32),
        ),
        scratch_types=[
            pltpu.VMEM((_BPW,), jnp.int32),
            pltpu.VMEM((_SUB,), jnp.int32),
            pltpu.VMEM((_SUB,), jnp.int32),
            pltpu.VMEM((_SUB,), jnp.int32),
            pltpu.VMEM((_SUB,), jnp.float32),
            pltpu.VMEM((_SUB,), jnp.float32),
            pltpu.VMEM((_SUB, _DIMP), jnp.float32),
            pltpu.VMEM((_SUB, _DIMP), jnp.float32),
            pltpu.SemaphoreType.DMA,
        ],
    )
    def gather_kernel(x_hbm, table_hbm, imp0_hbm, imp1_hbm,
                      t0_hbm, t1_hbm, w0_hbm, w1_hbm,
                      x_v, h0_v, h1_v, xs_v, i0_v, i1_v,
                      t0_v, t1_v, sem):
        wid = lax.axis_index("s") * _NC + lax.axis_index("c")
        base = wid * _BPW
        pltpu.sync_copy(x_hbm.at[pl.ds(base, _BPW)], x_v)

        for s in range(_BPW // _SUB):
            def hash_body(i, carry, s=s):
                xx = x_v[pl.ds(s * _SUB + i * _LANES, _LANES)]
                sl = pl.ds(i * _LANES, _LANES)
                h0_v[sl] = (xx * _PRIME0 + _OFF0) % _NUM_BUCKETS
                h1_v[sl] = (xx * _PRIME1 + _OFF1) % _NUM_BUCKETS
                xs_v[sl] = xx
                return carry

            lax.fori_loop(0, _SUB // _LANES, hash_body, 0)

            cp0 = pltpu.async_copy(table_hbm.at[h0_v], t0_v, sem)
            cp1 = pltpu.async_copy(table_hbm.at[h1_v], t1_v, sem)
            cp2 = pltpu.async_copy(imp0_hbm.at[xs_v], i0_v, sem)
            cp3 = pltpu.async_copy(imp1_hbm.at[xs_v], i1_v, sem)
            cp0.wait()
            cp1.wait()
            cp2.wait()
            cp3.wait()

            out_sl = pl.ds(base + s * _SUB, _SUB)
            pltpu.sync_copy(t0_v, t0_hbm.at[out_sl])
            pltpu.sync_copy(t1_v, t1_hbm.at[out_sl])
            pltpu.sync_copy(i0_v, w0_hbm.at[out_sl])
            pltpu.sync_copy(i1_v, w1_hbm.at[out_sl])

    return gather_kernel(x, table_p, imp0, imp1)


_PAD_BLK = 2000


def _pad_body(t_ref, o_ref):
    o_ref[:, : _DIM] = t_ref[...]
    o_ref[:, _DIM:] = jnp.zeros((_PAD_BLK, _DIMP - _DIM), jnp.float32)


def _pad_table(table):
    nblk = _NUM_BUCKETS // _PAD_BLK
    return pl.pallas_call(
        _pad_body,
        grid=(nblk,),
        in_specs=[pl.BlockSpec((_PAD_BLK, _DIM), lambda i: (i, 0))],
        out_specs=pl.BlockSpec((_PAD_BLK, _DIMP), lambda i: (i, 0)),
        out_shape=jax.ShapeDtypeStruct((_NUM_BUCKETS, _DIMP), jnp.float32),
    )(table)


_TC_BLK = 1024


def _tc_head_body(t0_ref, t1_ref, iw_ref, w_ref, b_ref, o_ref):
    t0 = t0_ref[...]
    t1 = t1_ref[...]
    iw = iw_ref[...]
    w0 = iw[:, 0:1]
    w1 = iw[:, 1:2]
    emb = w0 * t0 + w1 * t1
    out = jnp.dot(emb, w_ref[...], preferred_element_type=jnp.float32)
    out = out + b_ref[...]
    m = jnp.max(out, axis=1, keepdims=True)
    e = jnp.exp(out - m)
    s = jnp.sum(e, axis=1, keepdims=True)
    o_ref[...] = out - m - jnp.log(s)


def _tc_head(t0p, t1p, iw, fc_wp, fc_b2):
    nblk = _BATCH // _TC_BLK
    return pl.pallas_call(
        _tc_head_body,
        grid=(nblk,),
        in_specs=[
            pl.BlockSpec((_TC_BLK, _DIMP), lambda i: (i, 0)),
            pl.BlockSpec((_TC_BLK, _DIMP), lambda i: (i, 0)),
            pl.BlockSpec((_TC_BLK, 2), lambda i: (i, 0)),
            pl.BlockSpec((_DIMP, _OUT), lambda i: (0, 0)),
            pl.BlockSpec((1, _OUT), lambda i: (0, 0)),
        ],
        out_specs=pl.BlockSpec((_TC_BLK, _OUT), lambda i: (i, 0)),
        out_shape=jax.ShapeDtypeStruct((_BATCH, _OUT), jnp.float32),
    )(t0p, t1p, iw, fc_wp, fc_b2)


def kernel(x, table, importance, fc_w, fc_b):
    table_p = _pad_table(table)
    imp0 = importance[:, 0]
    imp1 = importance[:, 1]
    fc_wp = jnp.pad(fc_w, ((0, _DIMP - _DIM), (0, 0)))
    t0p, t1p, w0, w1 = _sc_gather(x, table_p, imp0, imp1)
    iw = jnp.stack([w0, w1], axis=1)
    return _tc_head(t0p, t1p, iw, fc_wp, fc_b.reshape(1, _OUT))


# speedup vs baseline: 11.0747x; 1.3715x over previous
import functools

import jax
import jax.numpy as jnp
from jax import lax
from jax.experimental import pallas as pl
from jax.experimental.pallas import tpu as pltpu
from jax.experimental.pallas import tpu_sc as plsc

_NUM_BUCKETS = 100000
_DIM = 100
_DIMP = 128
_PRIME0, _PRIME1 = 31, 1009
_OFF0, _OFF1 = 7, 433
_BATCH = 16384
_OUT = 300

_NC = 2
_NS = 16
_NW = _NC * _NS
_BPW = _BATCH // _NW
_SUB = 256
_LANES = 16


def _sc_gather(x, table_p, imp0, imp1):
    mesh = plsc.VectorSubcoreMesh(core_axis_name="c", subcore_axis_name="s")

    @functools.partial(
        pl.kernel,
        mesh=mesh,
        out_type=(
            jax.ShapeDtypeStruct((_BATCH, _DIMP), jnp.float32),
            jax.ShapeDtypeStruct((_BATCH, _DIMP), jnp.float32),
            jax.ShapeDtypeStruct((_BATCH,), jnp.float32),
            jax.ShapeDtypeStruct((_BATCH,), jnp.float32),
        ),
        scratch_types=[
            pltpu.VMEM((_BPW,), jnp.int32),
            pltpu.VMEM((_SUB,), jnp.int32),
            pltpu.VMEM((_SUB,), jnp.int32),
            pltpu.VMEM((_SUB,), jnp.int32),
            pltpu.VMEM((_SUB,), jnp.float32),
            pltpu.VMEM((_SUB,), jnp.float32),
            pltpu.VMEM((_SUB, _DIMP), jnp.float32),
            pltpu.VMEM((_SUB, _DIMP), jnp.float32),
            pltpu.SemaphoreType.DMA,
        ],
    )
    def gather_kernel(x_hbm, table_hbm, imp0_hbm, imp1_hbm,
                      t0_hbm, t1_hbm, w0_hbm, w1_hbm,
                      x_v, h0_v, h1_v, xs_v, i0_v, i1_v,
                      t0_v, t1_v, sem):
        wid = lax.axis_index("s") * _NC + lax.axis_index("c")
        base = wid * _BPW
        pltpu.sync_copy(x_hbm.at[pl.ds(base, _BPW)], x_v)

        for s in range(_BPW // _SUB):
            def hash_body(i, carry, s=s):
                xx = x_v[pl.ds(s * _SUB + i * _LANES, _LANES)]
                sl = pl.ds(i * _LANES, _LANES)
                h0_v[sl] = (xx * _PRIME0 + _OFF0) % _NUM_BUCKETS
                h1_v[sl] = (xx * _PRIME1 + _OFF1) % _NUM_BUCKETS
                xs_v[sl] = xx
                return carry

            lax.fori_loop(0, _SUB // _LANES, hash_body, 0)

            cp0 = pltpu.async_copy(table_hbm.at[h0_v], t0_v, sem)
            cp1 = pltpu.async_copy(table_hbm.at[h1_v], t1_v, sem)
            cp2 = pltpu.async_copy(imp0_hbm.at[xs_v], i0_v, sem)
            cp3 = pltpu.async_copy(imp1_hbm.at[xs_v], i1_v, sem)
            cp0.wait()
            cp1.wait()
            cp2.wait()
            cp3.wait()

            out_sl = pl.ds(base + s * _SUB, _SUB)
            pltpu.sync_copy(t0_v, t0_hbm.at[out_sl])
            pltpu.sync_copy(t1_v, t1_hbm.at[out_sl])
            pltpu.sync_copy(i0_v, w0_hbm.at[out_sl])
            pltpu.sync_copy(i1_v, w1_hbm.at[out_sl])

    return gather_kernel(x, table_p, imp0, imp1)


_PAD_BLK = 2048


def _pad_body(t_ref, o_ref):
    o_ref[:, : _DIM] = t_ref[...].T
    o_ref[:, _DIM:] = jnp.zeros((_PAD_BLK, _DIMP - _DIM), jnp.float32)


def _pad_table(table_t):
    nblk = (_NUM_BUCKETS + _PAD_BLK - 1) // _PAD_BLK
    return pl.pallas_call(
        _pad_body,
        grid=(nblk,),
        in_specs=[pl.BlockSpec((_DIM, _PAD_BLK), lambda i: (0, i))],
        out_specs=pl.BlockSpec((_PAD_BLK, _DIMP), lambda i: (i, 0)),
        out_shape=jax.ShapeDtypeStruct((_NUM_BUCKETS, _DIMP), jnp.float32),
    )(table_t)


_TC_BLK = 1024


def _tc_head_body(t0_ref, t1_ref, iw_ref, w_ref, b_ref, o_ref):
    t0 = t0_ref[...]
    t1 = t1_ref[...]
    iw = iw_ref[...]
    w0 = iw[:, 0:1]
    w1 = iw[:, 1:2]
    emb = w0 * t0 + w1 * t1
    out = jnp.dot(emb, w_ref[...], preferred_element_type=jnp.float32)
    out = out + b_ref[...]
    m = jnp.max(out, axis=1, keepdims=True)
    e = jnp.exp(out - m)
    s = jnp.sum(e, axis=1, keepdims=True)
    o_ref[...] = (out - m - jnp.log(s)).T


def _tc_head(t0p, t1p, iw, fc_wp, fc_b2):
    nblk = _BATCH // _TC_BLK
    return pl.pallas_call(
        _tc_head_body,
        grid=(nblk,),
        in_specs=[
            pl.BlockSpec((_TC_BLK, _DIMP), lambda i: (i, 0)),
            pl.BlockSpec((_TC_BLK, _DIMP), lambda i: (i, 0)),
            pl.BlockSpec((_TC_BLK, 2), lambda i: (i, 0)),
            pl.BlockSpec((_DIMP, _OUT), lambda i: (0, 0)),
            pl.BlockSpec((1, _OUT), lambda i: (0, 0)),
        ],
        out_specs=pl.BlockSpec((_OUT, _TC_BLK), lambda i: (0, i)),
        out_shape=jax.ShapeDtypeStruct((_OUT, _BATCH), jnp.float32),
    )(t0p, t1p, iw, fc_wp, fc_b2)


def kernel(x, table, importance, fc_w, fc_b):
    table_p = _pad_table(table.T)
    imp0 = importance[:, 0]
    imp1 = importance[:, 1]
    fc_wp = jnp.pad(fc_w, ((0, _DIMP - _DIM), (0, 0)))
    t0p, t1p, w0, w1 = _sc_gather(x, table_p, imp0, imp1)
    iw = jnp.stack([w0, w1], axis=1)
    out_t = _tc_head(t0p, t1p, iw, fc_wp, fc_b.reshape(1, _OUT))
    return out_t.T


# speedup vs baseline: 13.4066x; 1.2106x over previous
import functools

import jax
import jax.numpy as jnp
from jax import lax
from jax.experimental import pallas as pl
from jax.experimental.pallas import tpu as pltpu
from jax.experimental.pallas import tpu_sc as plsc

_NUM_BUCKETS = 100000
_DIM = 100
_DIMP = 128
_PRIME0, _PRIME1 = 31, 1009
_OFF0, _OFF1 = 7, 433
_BATCH = 16384
_OUT = 300

_NC = 2
_NS = 16
_NW = _NC * _NS
_BPW = _BATCH // _NW
_SUB = 256
_LANES = 16

_sc_mesh = plsc.VectorSubcoreMesh(core_axis_name="c", subcore_axis_name="s")


def _sc_gather_tables(x, table_p):
    @functools.partial(
        pl.kernel,
        mesh=_sc_mesh,
        out_type=(
            jax.ShapeDtypeStruct((_BATCH, _DIMP), jnp.float32),
            jax.ShapeDtypeStruct((_BATCH, _DIMP), jnp.float32),
        ),
        scratch_types=[
            pltpu.VMEM((_BPW,), jnp.int32),
            pltpu.VMEM((_SUB,), jnp.int32),
            pltpu.VMEM((_SUB,), jnp.int32),
            pltpu.VMEM((_SUB, _DIMP), jnp.float32),
            pltpu.VMEM((_SUB, _DIMP), jnp.float32),
            pltpu.SemaphoreType.DMA,
        ],
    )
    def tab_kernel(x_hbm, table_hbm, t0_hbm, t1_hbm,
                   x_v, h0_v, h1_v, t0_v, t1_v, sem):
        wid = lax.axis_index("s") * _NC + lax.axis_index("c")
        base = wid * _BPW
        pltpu.sync_copy(x_hbm.at[pl.ds(base, _BPW)], x_v)

        for s in range(_BPW // _SUB):
            def hash_body(i, carry, s=s):
                xx = x_v[pl.ds(s * _SUB + i * _LANES, _LANES)]
                sl = pl.ds(i * _LANES, _LANES)
                h0_v[sl] = (xx * _PRIME0 + _OFF0) % _NUM_BUCKETS
                h1_v[sl] = (xx * _PRIME1 + _OFF1) % _NUM_BUCKETS
                return carry

            lax.fori_loop(0, _SUB // _LANES, hash_body, 0)

            cp0 = pltpu.async_copy(table_hbm.at[h0_v], t0_v, sem)
            cp1 = pltpu.async_copy(table_hbm.at[h1_v], t1_v, sem)
            cp0.wait()
            cp1.wait()

            out_sl = pl.ds(base + s * _SUB, _SUB)
            pltpu.sync_copy(t0_v, t0_hbm.at[out_sl])
            pltpu.sync_copy(t1_v, t1_hbm.at[out_sl])

    return tab_kernel(x, table_p)


def _sc_gather_imp(x, imp0, imp1):
    @functools.partial(
        pl.kernel,
        mesh=_sc_mesh,
        out_type=(
            jax.ShapeDtypeStruct((_BATCH,), jnp.float32),
            jax.ShapeDtypeStruct((_BATCH,), jnp.float32),
        ),
        scratch_types=[
            pltpu.VMEM((_BPW,), jnp.int32),
            pltpu.VMEM((_BPW,), jnp.float32),
            pltpu.VMEM((_BPW,), jnp.float32),
            pltpu.SemaphoreType.DMA,
        ],
    )
    def imp_kernel(x_hbm, imp0_hbm, imp1_hbm, w0_hbm, w1_hbm,
                   x_v, i0_v, i1_v, sem):
        wid = lax.axis_index("s") * _NC + lax.axis_index("c")
        base = wid * _BPW
        sl = pl.ds(base, _BPW)
        pltpu.sync_copy(x_hbm.at[sl], x_v)
        cp0 = pltpu.async_copy(imp0_hbm.at[x_v], i0_v, sem)
        cp1 = pltpu.async_copy(imp1_hbm.at[x_v], i1_v, sem)
        cp0.wait()
        cp1.wait()
        pltpu.sync_copy(i0_v, w0_hbm.at[sl])
        pltpu.sync_copy(i1_v, w1_hbm.at[sl])

    return imp_kernel(x, imp0, imp1)


_PAD_BLK = 8192


def _pad_body(t_ref, o_ref):
    o_ref[:, : _DIM] = t_ref[...].T
    o_ref[:, _DIM:] = jnp.zeros((_PAD_BLK, _DIMP - _DIM), jnp.float32)


def _pad_table(table_t):
    nblk = (_NUM_BUCKETS + _PAD_BLK - 1) // _PAD_BLK
    return pl.pallas_call(
        _pad_body,
        grid=(nblk,),
        in_specs=[pl.BlockSpec((_DIM, _PAD_BLK), lambda i: (0, i))],
        out_specs=pl.BlockSpec((_PAD_BLK, _DIMP), lambda i: (i, 0)),
        out_shape=jax.ShapeDtypeStruct((_NUM_BUCKETS, _DIMP), jnp.float32),
    )(table_t)


_TC_BLK = 2048


def _tc_head_body(t0_ref, t1_ref, iw_ref, w_ref, b_ref, o_ref):
    t0 = t0_ref[...]
    t1 = t1_ref[...]
    iw = iw_ref[...]
    w0 = iw[:, 0:1]
    w1 = iw[:, 1:2]
    emb = w0 * t0 + w1 * t1
    out = jnp.dot(emb, w_ref[...], preferred_element_type=jnp.float32)
    out = out + b_ref[...]
    m = jnp.max(out, axis=1, keepdims=True)
    e = jnp.exp(out - m)
    s = jnp.sum(e, axis=1, keepdims=True)
    o_ref[...] = (out - m - jnp.log(s)).T


def _tc_head(t0p, t1p, iw, fc_wp, fc_b2):
    nblk = _BATCH // _TC_BLK
    return pl.pallas_call(
        _tc_head_body,
        grid=(nblk,),
        in_specs=[
            pl.BlockSpec((_TC_BLK, _DIMP), lambda i: (i, 0)),
            pl.BlockSpec((_TC_BLK, _DIMP), lambda i: (i, 0)),
            pl.BlockSpec((_TC_BLK, 2), lambda i: (i, 0)),
            pl.BlockSpec((_DIMP, _OUT), lambda i: (0, 0)),
            pl.BlockSpec((1, _OUT), lambda i: (0, 0)),
        ],
        out_specs=pl.BlockSpec((_OUT, _TC_BLK), lambda i: (0, i)),
        out_shape=jax.ShapeDtypeStruct((_OUT, _BATCH), jnp.float32),
    )(t0p, t1p, iw, fc_wp, fc_b2)


def kernel(x, table, importance, fc_w, fc_b):
    table_p = _pad_table(table.T)
    imp_t = importance.T
    imp0 = imp_t[0]
    imp1 = imp_t[1]
    fc_wp = jnp.pad(fc_w, ((0, _DIMP - _DIM), (0, 0)))
    t0p, t1p = _sc_gather_tables(x, table_p)
    w0, w1 = _sc_gather_imp(x, imp0, imp1)
    iw = jnp.stack([w0, w1], axis=1)
    out_t = _tc_head(t0p, t1p, iw, fc_wp, fc_b.reshape(1, _OUT))
    return out_t.T
